# Initial kernel scaffold; baseline (speedup 1.0000x reference)
import jax
import jax.numpy as jnp
from jax.experimental import pallas as pl


def kernel(x, edge_index, edge_attr, Wx, bx, We, be):
    raise NotImplementedError("write your pallas kernel here")


# speedup vs baseline: 1.5005x; 1.5005x over previous
import functools

import jax
import jax.numpy as jnp
from jax import lax
from jax.experimental import pallas as pl
from jax.experimental.pallas import tpu as pltpu
from jax.experimental.pallas import tpu_sc as plsc

N = 10000
E = 160000
D = 256
DE = 16
DO = D + DE

NC = 2
NS = 16
L = 16
NW = NC * NS
EPW = E // NW
B = 128
NFULL = EPW // B
TAIL = EPW - NFULL * B


def _mm_kernel(x_ref, w_ref, b_ref, o_ref, *, bias_scale):
    acc = lax.dot_general(
        x_ref[...], w_ref[...], (((1,), (1,)), ((), ())),
        preferred_element_type=jnp.float32,
    )
    o_ref[...] = acc + bias_scale * b_ref[...]


def _matmul_bias(x, w, b, block_m, bias_scale):
    m, k = x.shape
    n = w.shape[0]
    grid = m // block_m
    return pl.pallas_call(
        functools.partial(_mm_kernel, bias_scale=bias_scale),
        grid=(grid,),
        in_specs=[
            pl.BlockSpec((block_m, k), lambda i: (i, 0)),
            pl.BlockSpec((n, k), lambda i: (0, 0)),
            pl.BlockSpec((1, n), lambda i: (0, 0)),
        ],
        out_specs=pl.BlockSpec((block_m, n), lambda i: (i, 0)),
        out_shape=jax.ShapeDtypeStruct((m, n), jnp.float32),
    )(x, w, b.reshape(1, n))


_sc_mesh = plsc.VectorSubcoreMesh(core_axis_name="c", subcore_axis_name="s")


@functools.partial(
    pl.kernel,
    out_type=jax.ShapeDtypeStruct((E, DO), jnp.float32),
    mesh=_sc_mesh,
    scratch_types=[
        pltpu.VMEM((B,), jnp.int32),
        pltpu.VMEM((B,), jnp.int32),
        pltpu.VMEM((B, D), jnp.float32),
        pltpu.VMEM((B, D), jnp.float32),
        pltpu.VMEM((B, DE), jnp.float32),
        pltpu.SemaphoreType.DMA,
        pltpu.SemaphoreType.DMA,
        pltpu.SemaphoreType.DMA,
    ],
)
def _sc_edge_kernel(y_hbm, src_hbm, dst_hbm, ew_hbm, out_hbm,
                    isv, idv, ra, rb, re, s1, s2, s3):
    wid = lax.axis_index("s") * NC + lax.axis_index("c")
    base_w = wid * EPW

    def do_block(base, nb):
        base = pl.multiple_of(base, 8)
        is_ = isv.at[pl.ds(0, nb)]
        id_ = idv.at[pl.ds(0, nb)]
        ra_ = ra.at[pl.ds(0, nb)]
        rb_ = rb.at[pl.ds(0, nb)]
        re_ = re.at[pl.ds(0, nb)]
        pltpu.sync_copy(src_hbm.at[pl.ds(base, nb)], is_)
        pltpu.sync_copy(dst_hbm.at[pl.ds(base, nb)], id_)
        ca = pltpu.async_copy(y_hbm.at[is_], ra_, s1)
        cb = pltpu.async_copy(y_hbm.at[id_], rb_, s2)
        ce = pltpu.async_copy(ew_hbm.at[pl.ds(base, nb)], re_, s3)
        ca.wait()
        cb.wait()
        ce.wait()

        def addrow(r, carry):
            for c in range(D // L):
                sl = pl.ds(c * L, L)
                ra[r, sl] = ra[r, sl] + rb[r, sl]
            return carry

        lax.fori_loop(0, nb, addrow, 0)
        pltpu.sync_copy(ra_, out_hbm.at[pl.ds(base, nb), pl.ds(0, D)])
        pltpu.sync_copy(re_, out_hbm.at[pl.ds(base, nb), pl.ds(D, DE)])

    def blk(k, carry):
        do_block(base_w + k * B, B)
        return carry

    lax.fori_loop(0, NFULL, blk, 0)
    do_block(base_w + NFULL * B, TAIL)


def kernel(x, edge_index, edge_attr, Wx, bx, We, be):
    y = _matmul_bias(x, Wx, bx, block_m=2000, bias_scale=0.5)
    ew = _matmul_bias(edge_attr, We, be, block_m=8000, bias_scale=1.0)
    src = edge_index[0].astype(jnp.int32)
    dst = edge_index[1].astype(jnp.int32)
    return _sc_edge_kernel(y, src, dst, ew)


# speedup vs baseline: 1.8870x; 1.2576x over previous
import functools

import jax
import jax.numpy as jnp
from jax import lax
from jax.experimental import pallas as pl
from jax.experimental.pallas import tpu as pltpu
from jax.experimental.pallas import tpu_sc as plsc

N = 10000
E = 160000
D = 256
DE = 16
DO = D + DE

NC = 2
NS = 16
L = 16
NW = NC * NS
EPW = E // NW
B = 96
NB = EPW // B
TAIL = EPW - NB * B


def _mm_kernel(x_ref, w_ref, b_ref, o_ref, *, bias_scale):
    acc = lax.dot_general(
        x_ref[...], w_ref[...], (((1,), (1,)), ((), ())),
        preferred_element_type=jnp.float32,
    )
    o_ref[...] = acc + bias_scale * b_ref[...]


def _matmul_bias(x, w, b, block_m, bias_scale):
    m, k = x.shape
    n = w.shape[0]
    grid = m // block_m
    return pl.pallas_call(
        functools.partial(_mm_kernel, bias_scale=bias_scale),
        grid=(grid,),
        in_specs=[
            pl.BlockSpec((block_m, k), lambda i: (i, 0)),
            pl.BlockSpec((n, k), lambda i: (0, 0)),
            pl.BlockSpec((1, n), lambda i: (0, 0)),
        ],
        out_specs=pl.BlockSpec((block_m, n), lambda i: (i, 0)),
        out_shape=jax.ShapeDtypeStruct((m, n), jnp.float32),
    )(x, w, b.reshape(1, n))


_sc_mesh = plsc.VectorSubcoreMesh(core_axis_name="c", subcore_axis_name="s")


@functools.partial(
    pl.kernel,
    out_type=jax.ShapeDtypeStruct((E, DO), jnp.float32),
    mesh=_sc_mesh,
    scratch_types=[
        pltpu.VMEM((2, B), jnp.int32),
        pltpu.VMEM((2, B), jnp.int32),
        pltpu.VMEM((2, B, D), jnp.float32),
        pltpu.VMEM((2, B, D), jnp.float32),
        pltpu.VMEM((2, B, DE), jnp.float32),
        pltpu.SemaphoreType.DMA,
        pltpu.SemaphoreType.DMA,
        pltpu.SemaphoreType.DMA,
        pltpu.SemaphoreType.DMA,
        pltpu.SemaphoreType.DMA,
        pltpu.SemaphoreType.DMA,
    ],
)
def _sc_edge_kernel(y_hbm, src_hbm, dst_hbm, ew_hbm, out_hbm,
                    isv, idv, ra, rb, re,
                    si0, si1, sg0, sg1, sw0, sw1):
    wid = lax.axis_index("s") * NC + lax.axis_index("c")
    base_w = wid * EPW
    sis = (si0, si1)
    sgs = (sg0, sg1)
    sws = (sw0, sw1)

    def idx_copy(k, p, sem):
        base = pl.multiple_of(base_w + k * B, 8)
        a = pltpu.async_copy(src_hbm.at[pl.ds(base, B)], isv.at[p], sem)
        b = pltpu.async_copy(dst_hbm.at[pl.ds(base, B)], idv.at[p], sem)
        return a, b

    def idx_wait(p, sem):
        a, b = (pltpu.make_async_copy(src_hbm.at[pl.ds(0, B)], isv.at[p], sem),
                pltpu.make_async_copy(dst_hbm.at[pl.ds(0, B)], idv.at[p], sem))
        a.wait()
        b.wait()

    def gather_issue(k, p, sem):
        base = pl.multiple_of(base_w + k * B, 8)
        pltpu.async_copy(y_hbm.at[isv.at[p]], ra.at[p], sem)
        pltpu.async_copy(y_hbm.at[idv.at[p]], rb.at[p], sem)
        pltpu.async_copy(ew_hbm.at[pl.ds(base, B)], re.at[p], sem)

    def gather_wait(p, sem):
        pltpu.make_async_copy(y_hbm.at[isv.at[p]], ra.at[p], sem).wait()
        pltpu.make_async_copy(y_hbm.at[idv.at[p]], rb.at[p], sem).wait()
        pltpu.make_async_copy(ew_hbm.at[pl.ds(0, B)], re.at[p], sem).wait()

    def write_issue(k, p, sem):
        base = pl.multiple_of(base_w + k * B, 8)
        pltpu.async_copy(ra.at[p], out_hbm.at[pl.ds(base, B), pl.ds(0, D)], sem)
        pltpu.async_copy(re.at[p], out_hbm.at[pl.ds(base, B), pl.ds(D, DE)], sem)

    def write_wait(p, sem):
        pltpu.make_async_copy(ra.at[p], out_hbm.at[pl.ds(0, B), pl.ds(0, D)], sem).wait()
        pltpu.make_async_copy(re.at[p], out_hbm.at[pl.ds(0, B), pl.ds(D, DE)], sem).wait()

    def add_block(p, nb):
        def addrow(r, carry):
            for c in range(D // L):
                sl = pl.ds(c * L, L)
                ra[p, r, sl] = ra[p, r, sl] + rb[p, r, sl]
            return carry
        lax.fori_loop(0, nb, addrow, 0, unroll=2)

    a, b = idx_copy(0, 0, si0)
    a.wait()
    b.wait()
    gather_issue(0, 0, sg0)
    idx_copy(1, 1, si1)

    def pair(i, carry):
        k0 = 2 * i
        for p in (0, 1):
            k = k0 + p
            q = 1 - p
            sem_g, sem_w = sgs[p], sws[p]
            @pl.when(k + 1 < NB)
            def _():
                idx_wait(q, sis[q])

                @pl.when(k >= 1)
                def _():
                    write_wait(q, sws[q])
                gather_issue(k + 1, q, sgs[q])

            gather_wait(p, sem_g)
            @pl.when(k + 2 < NB)
            def _():
                idx_copy(k + 2, p, sis[p])
            add_block(p, B)
            write_issue(k, p, sem_w)
        return carry

    lax.fori_loop(0, NB // 2, pair, 0)
    write_wait(0, sw0)
    write_wait(1, sw1)

    base = pl.multiple_of(base_w + NB * B, 8)
    pltpu.sync_copy(src_hbm.at[pl.ds(base, TAIL)], isv.at[0, pl.ds(0, TAIL)])
    pltpu.sync_copy(dst_hbm.at[pl.ds(base, TAIL)], idv.at[0, pl.ds(0, TAIL)])
    ca = pltpu.async_copy(y_hbm.at[isv.at[0, pl.ds(0, TAIL)]],
                          ra.at[0, pl.ds(0, TAIL)], sg0)
    cb = pltpu.async_copy(y_hbm.at[idv.at[0, pl.ds(0, TAIL)]],
                          rb.at[0, pl.ds(0, TAIL)], sg0)
    ce = pltpu.async_copy(ew_hbm.at[pl.ds(base, TAIL)],
                          re.at[0, pl.ds(0, TAIL)], sg0)
    ca.wait()
    cb.wait()
    ce.wait()
    add_block(0, TAIL)
    pltpu.sync_copy(ra.at[0, pl.ds(0, TAIL)],
                    out_hbm.at[pl.ds(base, TAIL), pl.ds(0, D)])
    pltpu.sync_copy(re.at[0, pl.ds(0, TAIL)],
                    out_hbm.at[pl.ds(base, TAIL), pl.ds(D, DE)])


def kernel(x, edge_index, edge_attr, Wx, bx, We, be):
    y = _matmul_bias(x, Wx, bx, block_m=2000, bias_scale=0.5)
    ew = _matmul_bias(edge_attr, We, be, block_m=8000, bias_scale=1.0)
    src = edge_index[0].astype(jnp.int32)
    dst = edge_index[1].astype(jnp.int32)
    return _sc_edge_kernel(y, src, dst, ew)


# speedup vs baseline: 3.2234x; 1.7082x over previous
import functools

import jax
import jax.numpy as jnp
from jax import lax
from jax.experimental import pallas as pl
from jax.experimental.pallas import tpu as pltpu
from jax.experimental.pallas import tpu_sc as plsc

N = 10000
E = 160000
D = 256
DE = 16
DO = D + DE

NC = 2
NS = 16
L = 16
NW = NC * NS
EPW = E // NW
B = 112
NB = EPW // B
TAIL = EPW - NB * B

_sc_mesh = plsc.VectorSubcoreMesh(core_axis_name="c", subcore_axis_name="s")


@functools.partial(
    pl.kernel,
    out_type=jax.ShapeDtypeStruct((E, D), jnp.float32),
    mesh=_sc_mesh,
    scratch_types=[
        pltpu.VMEM((2, B), jnp.int32),
        pltpu.VMEM((2, B), jnp.int32),
        pltpu.VMEM((2, B, D), jnp.float32),
        pltpu.VMEM((2, B, D), jnp.float32),
        pltpu.SemaphoreType.DMA,
        pltpu.SemaphoreType.DMA,
        pltpu.SemaphoreType.DMA,
        pltpu.SemaphoreType.DMA,
        pltpu.SemaphoreType.DMA,
        pltpu.SemaphoreType.DMA,
    ],
)
def _sc_gather_add(x_hbm, src_hbm, dst_hbm, g_hbm,
                   isv, idv, ra, rb,
                   si0, si1, sg0, sg1, sw0, sw1):
    wid = lax.axis_index("s") * NC + lax.axis_index("c")
    base_w = wid * EPW
    sis = (si0, si1)
    sgs = (sg0, sg1)
    sws = (sw0, sw1)

    def idx_copy(k, p, sem):
        base = pl.multiple_of(base_w + k * B, 8)
        pltpu.async_copy(src_hbm.at[pl.ds(base, B)], isv.at[p], sem)
        pltpu.async_copy(dst_hbm.at[pl.ds(base, B)], idv.at[p], sem)

    def idx_wait(p, sem):
        pltpu.make_async_copy(src_hbm.at[pl.ds(0, B)], isv.at[p], sem).wait()
        pltpu.make_async_copy(dst_hbm.at[pl.ds(0, B)], idv.at[p], sem).wait()

    def gather_issue(p):
        pltpu.async_copy(x_hbm.at[isv.at[p]], ra.at[p], sgs[p])
        pltpu.async_copy(x_hbm.at[idv.at[p]], rb.at[p], sgs[p])

    def gather_wait(p):
        pltpu.make_async_copy(x_hbm.at[isv.at[p]], ra.at[p], sgs[p]).wait()
        pltpu.make_async_copy(x_hbm.at[idv.at[p]], rb.at[p], sgs[p]).wait()

    def write_issue(k, p):
        base = pl.multiple_of(base_w + k * B, 8)
        pltpu.async_copy(ra.at[p], g_hbm.at[pl.ds(base, B)], sws[p])

    def write_wait(p):
        pltpu.make_async_copy(ra.at[p], g_hbm.at[pl.ds(0, B)], sws[p]).wait()

    def add_block(p, nb):
        def addrow(r, carry):
            for c in range(D // L):
                sl = pl.ds(c * L, L)
                ra[p, r, sl] = ra[p, r, sl] + rb[p, r, sl]
            return carry
        lax.fori_loop(0, nb, addrow, 0, unroll=2)

    idx_copy(0, 0, si0)
    idx_wait(0, si0)
    gather_issue(0)
    idx_copy(1, 1, si1)

    def pair(i, carry):
        k0 = 2 * i
        for p in (0, 1):
            k = k0 + p
            q = 1 - p
            @pl.when(k + 1 < NB)
            def _():
                idx_wait(q, sis[q])

                @pl.when(k >= 1)
                def _():
                    write_wait(q)
                gather_issue(q)

            gather_wait(p)
            @pl.when(k + 2 < NB)
            def _():
                idx_copy(k + 2, p, sis[p])
            add_block(p, B)
            write_issue(k, p)
        return carry

    lax.fori_loop(0, NB // 2, pair, 0)
    write_wait(0)
    write_wait(1)

    base = pl.multiple_of(base_w + NB * B, 8)
    pltpu.sync_copy(src_hbm.at[pl.ds(base, TAIL)], isv.at[0, pl.ds(0, TAIL)])
    pltpu.sync_copy(dst_hbm.at[pl.ds(base, TAIL)], idv.at[0, pl.ds(0, TAIL)])
    ca = pltpu.async_copy(x_hbm.at[isv.at[0, pl.ds(0, TAIL)]],
                          ra.at[0, pl.ds(0, TAIL)], sg0)
    cb = pltpu.async_copy(x_hbm.at[idv.at[0, pl.ds(0, TAIL)]],
                          rb.at[0, pl.ds(0, TAIL)], sg0)
    ca.wait()
    cb.wait()
    add_block(0, TAIL)
    pltpu.sync_copy(ra.at[0, pl.ds(0, TAIL)], g_hbm.at[pl.ds(base, TAIL)])


BK = 3200


def _tc_out_body(g_ref, wx_ref, bx_ref, eaT_ref, we_ref, be_ref, o_ref):
    h = lax.dot_general(wx_ref[...], g_ref[...], (((1,), (1,)), ((), ())),
                        preferred_element_type=jnp.float32)
    o_ref[0:D, :] = h + bx_ref[...]
    e = lax.dot_general(we_ref[...], eaT_ref[...], (((1,), (0,)), ((), ())),
                        preferred_element_type=jnp.float32)
    o_ref[D:DO, :] = e + be_ref[...]


def _tc_out(g, Wx, bx, eaT, We, be):
    return pl.pallas_call(
        _tc_out_body,
        grid=(E // BK,),
        in_specs=[
            pl.BlockSpec((BK, D), lambda i: (i, 0)),
            pl.BlockSpec((D, D), lambda i: (0, 0)),
            pl.BlockSpec((D, 1), lambda i: (0, 0)),
            pl.BlockSpec((DE, BK), lambda i: (0, i)),
            pl.BlockSpec((DE, DE), lambda i: (0, 0)),
            pl.BlockSpec((DE, 1), lambda i: (0, 0)),
        ],
        out_specs=pl.BlockSpec((DO, BK), lambda i: (0, i)),
        out_shape=jax.ShapeDtypeStruct((DO, E), jnp.float32),
    )(g, Wx, bx.reshape(D, 1), eaT, We, be.reshape(DE, 1))


def kernel(x, edge_index, edge_attr, Wx, bx, We, be):
    src = edge_index[0].astype(jnp.int32)
    dst = edge_index[1].astype(jnp.int32)
    g = _sc_gather_add(x, src, dst)
    outT = _tc_out(g, Wx, bx, edge_attr.T, We, be)
    return outT.T


# speedup vs baseline: 3.2300x; 1.0020x over previous
import functools

import jax
import jax.numpy as jnp
from jax import lax
from jax.experimental import pallas as pl
from jax.experimental.pallas import tpu as pltpu
from jax.experimental.pallas import tpu_sc as plsc

N = 10000
E = 160000
D = 256
DE = 16
DO = D + DE

E1 = 76800
E2 = E - E1

NC = 2
NS = 16
L = 16
NW = NC * NS
B = 80

_sc_mesh = plsc.VectorSubcoreMesh(core_axis_name="c", subcore_axis_name="s")


def _make_sc_gather_add(e_chunk):
    epw = e_chunk // NW
    nb = (epw // B) & ~1
    tail = epw - nb * B
    assert epw % 8 == 0 and tail % 8 == 0

    @functools.partial(
        pl.kernel,
        out_type=jax.ShapeDtypeStruct((e_chunk, D), jnp.float32),
        mesh=_sc_mesh,
        scratch_types=[
            pltpu.VMEM((2, B), jnp.int32),
            pltpu.VMEM((2, B), jnp.int32),
            pltpu.VMEM((2, B, D), jnp.float32),
            pltpu.VMEM((2, B, D), jnp.float32),
            pltpu.SemaphoreType.DMA,
            pltpu.SemaphoreType.DMA,
            pltpu.SemaphoreType.DMA,
            pltpu.SemaphoreType.DMA,
            pltpu.SemaphoreType.DMA,
            pltpu.SemaphoreType.DMA,
        ],
    )
    def _sc_gather_add(x_hbm, src_hbm, dst_hbm, g_hbm,
                       isv, idv, ra, rb,
                       si0, si1, sg0, sg1, sw0, sw1):
        wid = lax.axis_index("s") * NC + lax.axis_index("c")
        base_w = wid * epw
        sis = (si0, si1)
        sgs = (sg0, sg1)
        sws = (sw0, sw1)

        def idx_copy(k, p, sem):
            base = pl.multiple_of(base_w + k * B, 8)
            pltpu.async_copy(src_hbm.at[pl.ds(base, B)], isv.at[p], sem)
            pltpu.async_copy(dst_hbm.at[pl.ds(base, B)], idv.at[p], sem)

        def idx_wait(p, sem):
            pltpu.make_async_copy(src_hbm.at[pl.ds(0, B)], isv.at[p], sem).wait()
            pltpu.make_async_copy(dst_hbm.at[pl.ds(0, B)], idv.at[p], sem).wait()

        def gather_issue(p):
            pltpu.async_copy(x_hbm.at[isv.at[p]], ra.at[p], sgs[p])
            pltpu.async_copy(x_hbm.at[idv.at[p]], rb.at[p], sgs[p])

        def gather_wait(p):
            pltpu.make_async_copy(x_hbm.at[isv.at[p]], ra.at[p], sgs[p]).wait()
            pltpu.make_async_copy(x_hbm.at[idv.at[p]], rb.at[p], sgs[p]).wait()

        def write_issue(k, p):
            base = pl.multiple_of(base_w + k * B, 8)
            pltpu.async_copy(ra.at[p], g_hbm.at[pl.ds(base, B)], sws[p])

        def write_wait(p):
            pltpu.make_async_copy(ra.at[p], g_hbm.at[pl.ds(0, B)], sws[p]).wait()

        def add_block(p, rows):
            def addrow(r, carry):
                for c in range(D // L):
                    sl = pl.ds(c * L, L)
                    ra[p, r, sl] = ra[p, r, sl] + rb[p, r, sl]
                return carry
            lax.fori_loop(0, rows, addrow, 0, unroll=2)

        idx_copy(0, 0, si0)
        idx_wait(0, si0)
        gather_issue(0)
        idx_copy(1, 1, si1)

        def pair(i, carry):
            k0 = 2 * i
            for p in (0, 1):
                k = k0 + p
                q = 1 - p
                @pl.when(k + 1 < nb)
                def _():
                    idx_wait(q, sis[q])

                    @pl.when(k >= 1)
                    def _():
                        write_wait(q)
                    gather_issue(q)

                gather_wait(p)
                @pl.when(k + 2 < nb)
                def _():
                    idx_copy(k + 2, p, sis[p])
                add_block(p, B)
                write_issue(k, p)
            return carry

        lax.fori_loop(0, nb // 2, pair, 0)
        write_wait(0)
        write_wait(1)

        done = nb * B
        while done < epw:
            t = min(B, epw - done)
            base = pl.multiple_of(base_w + done, 8)
            pltpu.sync_copy(src_hbm.at[pl.ds(base, t)], isv.at[0, pl.ds(0, t)])
            pltpu.sync_copy(dst_hbm.at[pl.ds(base, t)], idv.at[0, pl.ds(0, t)])
            ca = pltpu.async_copy(x_hbm.at[isv.at[0, pl.ds(0, t)]],
                                  ra.at[0, pl.ds(0, t)], sg0)
            cb = pltpu.async_copy(x_hbm.at[idv.at[0, pl.ds(0, t)]],
                                  rb.at[0, pl.ds(0, t)], sg0)
            ca.wait()
            cb.wait()
            add_block(0, t)
            pltpu.sync_copy(ra.at[0, pl.ds(0, t)], g_hbm.at[pl.ds(base, t)])
            done += t

    return _sc_gather_add


_sc_gather_add_1 = _make_sc_gather_add(E1)
_sc_gather_add_2 = _make_sc_gather_add(E2)


BK = 3200


def _tc_out_body(g_ref, wx_ref, bx_ref, eaT_ref, we_ref, be_ref, o_ref):
    h = lax.dot_general(wx_ref[...], g_ref[...], (((1,), (1,)), ((), ())),
                        preferred_element_type=jnp.float32)
    o_ref[0:D, :] = h + bx_ref[...]
    e = lax.dot_general(we_ref[...], eaT_ref[...], (((1,), (0,)), ((), ())),
                        preferred_element_type=jnp.float32)
    o_ref[D:DO, :] = e + be_ref[...]


def _tc_out_first(g, Wx, bx, eaT, We, be):
    return pl.pallas_call(
        _tc_out_body,
        grid=(E1 // BK,),
        in_specs=[
            pl.BlockSpec((BK, D), lambda i: (i, 0)),
            pl.BlockSpec((D, D), lambda i: (0, 0)),
            pl.BlockSpec((D, 1), lambda i: (0, 0)),
            pl.BlockSpec((DE, BK), lambda i: (0, i)),
            pl.BlockSpec((DE, DE), lambda i: (0, 0)),
            pl.BlockSpec((DE, 1), lambda i: (0, 0)),
        ],
        out_specs=pl.BlockSpec((DO, BK), lambda i: (0, i)),
        out_shape=jax.ShapeDtypeStruct((DO, E), jnp.float32),
    )(g, Wx, bx.reshape(D, 1), eaT, We, be.reshape(DE, 1))


def _tc_out_second(g, Wx, bx, eaT, We, be, outT):
    off = E1 // BK

    def body(g_ref, wx_ref, bx_ref, eaT_ref, we_ref, be_ref, prev_ref, o_ref):
        _tc_out_body(g_ref, wx_ref, bx_ref, eaT_ref, we_ref, be_ref, o_ref)

    return pl.pallas_call(
        body,
        grid=(E2 // BK,),
        in_specs=[
            pl.BlockSpec((BK, D), lambda i: (i, 0)),
            pl.BlockSpec((D, D), lambda i: (0, 0)),
            pl.BlockSpec((D, 1), lambda i: (0, 0)),
            pl.BlockSpec((DE, BK), lambda i: (0, i + off)),
            pl.BlockSpec((DE, DE), lambda i: (0, 0)),
            pl.BlockSpec((DE, 1), lambda i: (0, 0)),
            pl.BlockSpec(memory_space=pltpu.MemorySpace.HBM),
        ],
        out_specs=pl.BlockSpec((DO, BK), lambda i: (0, i + off)),
        out_shape=jax.ShapeDtypeStruct((DO, E), jnp.float32),
        input_output_aliases={6: 0},
    )(g, Wx, bx.reshape(D, 1), eaT, We, be.reshape(DE, 1), outT)


def kernel(x, edge_index, edge_attr, Wx, bx, We, be):
    src = edge_index[0].astype(jnp.int32)
    dst = edge_index[1].astype(jnp.int32)
    eaT = edge_attr.T
    g1 = _sc_gather_add_1(x, src[:E1], dst[:E1])
    g2 = _sc_gather_add_2(x, src[E1:], dst[E1:])
    o1 = _tc_out_first(g1, Wx, bx, eaT, We, be)
    outT = _tc_out_second(g2, Wx, bx, eaT, We, be, o1)
    return outT.T
